# baseline (speedup 1.0000x reference)
import functools

import jax
import jax.numpy as jnp
from jax import lax
from jax.experimental import pallas as pl
from jax.experimental.pallas import tpu as pltpu
from jax.experimental.pallas import tpu_sc as plsc

NC = 2
NS = 16
L = 16
NW = NC * NS

B = 16384
C = 1000
PER_W = B // NW
CHUNKS = PER_W // 128
ROWS128 = B // 128


def _gather_body(logits_hbm, y_hbm, out_hbm, y_v, idx_v, vals_v, sem):
    wid = lax.axis_index("s") * NC + lax.axis_index("c")
    row0 = wid * CHUNKS
    pltpu.sync_copy(y_hbm.at[pl.ds(row0, CHUNKS)], y_v)
    for r in range(CHUNKS):
        for g in range(128 // L):
            col = g * L
            sample0 = (row0 + r) * 128 + col
            yv = y_v[r, pl.ds(col, L)]
            rows = sample0 + lax.iota(jnp.int32, L)
            idx_v[r, pl.ds(col, L)] = rows * C + yv
    copies = [
        pltpu.async_copy(logits_hbm.at[idx_v.at[r]], vals_v.at[r], sem)
        for r in range(CHUNKS)
    ]
    for cp in copies:
        cp.wait()
    pltpu.sync_copy(vals_v, out_hbm.at[pl.ds(row0, CHUNKS)])


@jax.jit
def _identity_loss(logits_flat, y2d):
    mesh = plsc.VectorSubcoreMesh(core_axis_name="c", subcore_axis_name="s")
    run = pl.kernel(
        _gather_body,
        out_type=jax.ShapeDtypeStruct((ROWS128, 128), jnp.float32),
        mesh=mesh,
        scratch_types=[
            pltpu.VMEM((CHUNKS, 128), jnp.int32),
            pltpu.VMEM((CHUNKS, 128), jnp.int32),
            pltpu.VMEM((CHUNKS, 128), jnp.float32),
            pltpu.SemaphoreType.DMA,
        ],
    )
    return run(logits_flat, y2d)


def kernel(logits, y):
    logits_flat = logits.reshape(-1)
    y2d = y.astype(jnp.int32).reshape(ROWS128, 128)
    return _identity_loss(logits_flat, y2d).reshape(-1)


# speedup vs baseline: 7.0143x; 7.0143x over previous
import functools

import jax
import jax.numpy as jnp
from jax import lax
from jax.experimental import pallas as pl
from jax.experimental.pallas import tpu as pltpu
from jax.experimental.pallas import tpu_sc as plsc

NC = 2
NS = 16
L = 16
NW = NC * NS

B = 16384
C = 1000
PER_W = B // NW
CHUNKS = PER_W // 128
ROWS128 = B // 128


def _gather_body(logits_hbm, y_hbm, out_hbm, y_v, idx_v, vals_v, sem):
    wid = lax.axis_index("s") * NC + lax.axis_index("c")
    row0 = wid * CHUNKS
    pltpu.sync_copy(y_hbm.at[pl.ds(row0, CHUNKS)], y_v)
    for r in range(CHUNKS):
        b = row0 + r
        for g in range(128 // L):
            col = g * L
            yv = y_v[r, pl.ds(col, L)]
            lane = col + lax.iota(jnp.int32, L)
            idx_v[r, pl.ds(col, L)] = (
                (yv >> 3) * 131072 + b * 1024 + (yv & 7) * 128 + lane
            )
    copies = [
        pltpu.async_copy(logits_hbm.at[idx_v.at[r]], vals_v.at[r], sem)
        for r in range(CHUNKS)
    ]
    for cp in copies:
        cp.wait()
    pltpu.sync_copy(vals_v, out_hbm.at[pl.ds(row0, CHUNKS)])


@jax.jit
def _identity_loss(logits_flat, y2d):
    mesh = plsc.VectorSubcoreMesh(core_axis_name="c", subcore_axis_name="s")
    run = pl.kernel(
        _gather_body,
        out_type=jax.ShapeDtypeStruct((ROWS128, 128), jnp.float32),
        mesh=mesh,
        scratch_types=[
            pltpu.VMEM((CHUNKS, 128), jnp.int32),
            pltpu.VMEM((CHUNKS, 128), jnp.int32),
            pltpu.VMEM((CHUNKS, 128), jnp.float32),
            pltpu.SemaphoreType.DMA,
        ],
    )
    return run(logits_flat, y2d)


def kernel(logits, y):
    logits_flat = (
        logits.T.reshape(C // 8, 8, B // 128, 128)
        .transpose(0, 2, 1, 3)
        .reshape(-1)
    )
    y2d = y.astype(jnp.int32).reshape(ROWS128, 128)
    return _identity_loss(logits_flat, y2d).reshape(-1)
